# baseline (speedup 1.0000x reference)
import jax
import jax.numpy as jnp
from jax.experimental import pallas as pl
from jax.experimental.pallas import tpu as pltpu


def _moe_argmax_body(nblk, pc, x_ref, m_ref, o_ref, acc_ref, nrm_ref):
    k = pl.program_id(0)

    @pl.when(k == 0)
    def _init():
        acc_ref[...] = jnp.zeros_like(acc_ref)
        nrm_ref[...] = jnp.zeros_like(nrm_ref)

    acc = None
    nrm = None
    for i in range(pc):
        xi = x_ref[:, i, :]
        mi = m_ref[:, i, :]
        d = jax.lax.dot_general(
            xi, mi, (((1,), (1,)), ((), ())), preferred_element_type=jnp.float32
        )
        n = jnp.sum(mi * mi, axis=1, keepdims=True)
        acc = d if acc is None else acc + d
        nrm = n if nrm is None else nrm + n
    acc_ref[...] += acc
    nrm_ref[...] += nrm

    @pl.when(k == nblk - 1)
    def _finish():
        bn = jnp.maximum(jnp.sqrt(nrm_ref[...]), 1e-8)
        scores = acc_ref[...] * (1.0 / bn).T
        o_ref[...] = jnp.argmax(scores, axis=-1)[:, None].astype(jnp.int32)


def kernel(x, memory):
    b, h, w, d = x.shape
    e = memory.shape[0]
    p = h * w
    xv = x.reshape(b, p, d)
    mv = memory

    pc = 128
    nblk = p // pc

    out = pl.pallas_call(
        lambda *refs: _moe_argmax_body(nblk, pc, *refs),
        grid=(nblk,),
        in_specs=[
            pl.BlockSpec((b, pc, d), lambda k: (0, k, 0)),
            pl.BlockSpec((e, pc, d), lambda k: (0, k, 0)),
        ],
        out_specs=pl.BlockSpec((b, 1), lambda k: (0, 0)),
        out_shape=jax.ShapeDtypeStruct((b, 1), jnp.int32),
        scratch_shapes=[
            pltpu.VMEM((b, e), jnp.float32),
            pltpu.VMEM((e, 1), jnp.float32),
        ],
    )(xv, mv)
    return out.reshape(b)


# speedup vs baseline: 1.1133x; 1.1133x over previous
import jax
import jax.numpy as jnp
from jax.experimental import pallas as pl
from jax.experimental.pallas import tpu as pltpu


def _moe_argmax_body(nblk, pc, x_ref, m_ref, o_ref, acc_ref, nrm_ref):
    k = pl.program_id(0)

    @pl.when(k == 0)
    def _init():
        acc_ref[...] = jnp.zeros_like(acc_ref)
        nrm_ref[...] = jnp.zeros_like(nrm_ref)

    acc = None
    nrm = None
    for i in range(pc):
        xi = x_ref[:, i, :]
        mi = m_ref[:, i, :]
        d = jax.lax.dot_general(
            xi, mi, (((1,), (1,)), ((), ())), preferred_element_type=jnp.float32
        )
        n = jnp.sum(mi * mi, axis=1, keepdims=True)
        acc = d if acc is None else acc + d
        nrm = n if nrm is None else nrm + n
    acc_ref[...] += acc
    nrm_ref[...] += nrm

    @pl.when(k == nblk - 1)
    def _finish():
        bn = jnp.maximum(jnp.sqrt(nrm_ref[...]), 1e-8)
        scores = acc_ref[...] * (1.0 / bn).T
        o_ref[...] = jnp.argmax(scores, axis=-1)[:, None].astype(jnp.int32)


def kernel(x, memory):
    b, h, w, d = x.shape
    e = memory.shape[0]
    p = h * w
    xv = x.reshape(b, p, d)
    mv = memory

    pc = 64
    nblk = p // pc

    out = pl.pallas_call(
        lambda *refs: _moe_argmax_body(nblk, pc, *refs),
        grid=(nblk,),
        in_specs=[
            pl.BlockSpec((b, pc, d), lambda k: (0, k, 0)),
            pl.BlockSpec((e, pc, d), lambda k: (0, k, 0)),
        ],
        out_specs=pl.BlockSpec((b, 1), lambda k: (0, 0)),
        out_shape=jax.ShapeDtypeStruct((b, 1), jnp.int32),
        scratch_shapes=[
            pltpu.VMEM((b, e), jnp.float32),
            pltpu.VMEM((e, 1), jnp.float32),
        ],
    )(xv, mv)
    return out.reshape(b)
